# baseline (speedup 1.0000x reference)
import functools

import jax
import jax.numpy as jnp
from jax import lax
from jax.experimental import pallas as pl
from jax.experimental.pallas import tpu as pltpu
from jax.experimental.pallas import tpu_sc as plsc

SIGMA = 1.0
NUM_CORES = 2
NUM_SUBCORES = 16
NUM_WORKERS = NUM_CORES * NUM_SUBCORES
LANES = 16
CHUNK = 128


def _body(frame_num, latent_dim, b_per_w, style_hbm, frame_hbm, lat_hbm,
          mu_hbm, out_hbm, sid_v, fid_v, idx_v, lat_v, mu_v, sem):
    n_chunks = b_per_w // CHUNK
    wid = lax.axis_index("s") * NUM_CORES + lax.axis_index("c")
    base = wid * b_per_w

    for j in range(n_chunks):
        pltpu.sync_copy(style_hbm.at[pl.ds(base + j * CHUNK, CHUNK)],
                        sid_v.at[j])
        pltpu.sync_copy(frame_hbm.at[pl.ds(base + j * CHUNK, CHUNK)],
                        fid_v.at[j])

    def compute_idx(i, _):
        j = i // (CHUNK // LANES)
        k = (i % (CHUNK // LANES)) * LANES
        s = sid_v[j, pl.ds(k, LANES)]
        f = fid_v[j, pl.ds(k, LANES)]
        idx_v[j, pl.ds(k, LANES)] = s * frame_num + f
        return 0

    lax.fori_loop(0, n_chunks * (CHUNK // LANES), compute_idx, 0)

    copies = []
    for j in range(n_chunks):
        copies.append(pltpu.async_copy(lat_hbm.at[idx_v.at[j]],
                                       lat_v.at[j], sem))
        copies.append(pltpu.async_copy(mu_hbm.at[sid_v.at[j]],
                                       mu_v.at[j], sem))
    for c in copies:
        c.wait()

    def combine(r, _):
        for j in range(n_chunks):
            for d in range(latent_dim // LANES):
                lat = lat_v[j, r, pl.ds(d * LANES, LANES)]
                mu = mu_v[j, r, pl.ds(d * LANES, LANES)]
                lat_v[j, r, pl.ds(d * LANES, LANES)] = mu + SIGMA * (lat - mu)
        return 0

    lax.fori_loop(0, CHUNK, combine, 0)

    for j in range(n_chunks):
        pltpu.sync_copy(lat_v.at[j],
                        out_hbm.at[pl.ds(base + j * CHUNK, CHUNK)])


def kernel(style_ids, frame_ids, latents, style_latents_mu):
    style_num, frame_num, latent_dim = latents.shape
    batch = style_ids.shape[0]
    b_per_w = batch // NUM_WORKERS
    n_chunks = b_per_w // CHUNK
    lat_flat = latents.reshape(style_num * frame_num, latent_dim)

    mesh = plsc.VectorSubcoreMesh(core_axis_name="c", subcore_axis_name="s",
                                  num_cores=NUM_CORES,
                                  num_subcores=NUM_SUBCORES)
    run = pl.kernel(
        functools.partial(_body, frame_num, latent_dim, b_per_w),
        out_type=jax.ShapeDtypeStruct((batch, latent_dim), jnp.float32),
        mesh=mesh,
        scratch_types=[
            pltpu.VMEM((n_chunks, CHUNK), jnp.int32),
            pltpu.VMEM((n_chunks, CHUNK), jnp.int32),
            pltpu.VMEM((n_chunks, CHUNK), jnp.int32),
            pltpu.VMEM((n_chunks, CHUNK, latent_dim), jnp.float32),
            pltpu.VMEM((n_chunks, CHUNK, latent_dim), jnp.float32),
            pltpu.SemaphoreType.DMA,
        ],
        compiler_params=pltpu.CompilerParams(use_tc_tiling_on_sc=False),
    )
    return run(style_ids, frame_ids, lat_flat, style_latents_mu)


# speedup vs baseline: 1.7198x; 1.7198x over previous
import functools

import jax
import jax.numpy as jnp
from jax import lax
from jax.experimental import pallas as pl
from jax.experimental.pallas import tpu as pltpu
from jax.experimental.pallas import tpu_sc as plsc

SIGMA = 1.0
NUM_CORES = 2
NUM_SUBCORES = 16
NUM_WORKERS = NUM_CORES * NUM_SUBCORES
LANES = 16
FBLK = 128


def _body(latent_dim, b_per_w, style_hbm, frame_hbm, lat_hbm, mu_hbm,
          out_hbm, sid_s, fid_s, slab0, slab1, mur0, mur1, out_v,
          sem0, sem1):
    wid = lax.axis_index("s") * NUM_CORES + lax.axis_index("c")
    base = wid * b_per_w

    pltpu.sync_copy(style_hbm.at[pl.ds(base, b_per_w)],
                    sid_s.at[pl.ds(0, b_per_w)])
    pltpu.sync_copy(frame_hbm.at[pl.ds(base, b_per_w)],
                    fid_s.at[pl.ds(0, b_per_w)])

    def sid(b):
        return sid_s[pl.ds(b, LANES)][0]

    def fid(b):
        return fid_s[pl.ds(b, LANES)][0]

    def slab_src(b):
        off = pl.multiple_of((fid(b) // FBLK) * FBLK, FBLK)
        return lat_hbm.at[sid(b), :, pl.ds(off, FBLK)]

    def mu_src(b):
        return mu_hbm.at[sid(b), :]

    def fire(b, slab, mur, sem):
        pltpu.async_copy(slab_src(b), slab, sem)
        pltpu.async_copy(mu_src(b), mur, sem)

    def wait(b, slab, mur, sem):
        pltpu.make_async_copy(slab_src(b), slab, sem).wait()
        pltpu.make_async_copy(mu_src(b), mur, sem).wait()

    fire(0, slab0, mur0, sem0)

    d_iota = lax.broadcasted_iota(jnp.int32, (LANES,), 0)

    def step(b, _):
        @pl.when(b + 1 < b_per_w)
        def _prefetch():
            @pl.when((b + 1) % 2 == 0)
            def _():
                fire(b + 1, slab0, mur0, sem0)

            @pl.when((b + 1) % 2 == 1)
            def _():
                fire(b + 1, slab1, mur1, sem1)

        foff = fid(b) % FBLK
        col_idx = jnp.full((LANES,), foff, jnp.int32)

        for slab, mur, sem, par in ((slab0, mur0, sem0, 0),
                                    (slab1, mur1, sem1, 1)):
            @pl.when(b % 2 == par)
            def _consume(slab=slab, mur=mur, sem=sem):
                wait(b, slab, mur, sem)
                for d0 in range(0, latent_dim, LANES):
                    lat = plsc.load_gather(slab, [d0 + d_iota, col_idx])
                    mu = mur[pl.ds(d0, LANES)]
                    out_v[b, pl.ds(d0, LANES)] = mu + SIGMA * (lat - mu)

        return 0

    lax.fori_loop(0, b_per_w, step, 0)

    pltpu.sync_copy(out_v, out_hbm.at[pl.ds(base, b_per_w)])


def kernel(style_ids, frame_ids, latents, style_latents_mu):
    style_num, frame_num, latent_dim = latents.shape
    batch = style_ids.shape[0]
    b_per_w = batch // NUM_WORKERS
    lat_t = jnp.transpose(latents, (0, 2, 1))

    mesh = plsc.VectorSubcoreMesh(core_axis_name="c", subcore_axis_name="s",
                                  num_cores=NUM_CORES,
                                  num_subcores=NUM_SUBCORES)
    run = pl.kernel(
        functools.partial(_body, latent_dim, b_per_w),
        out_type=jax.ShapeDtypeStruct((batch, latent_dim), jnp.float32),
        mesh=mesh,
        scratch_types=[
            pltpu.VMEM((b_per_w + LANES,), jnp.int32),
            pltpu.VMEM((b_per_w + LANES,), jnp.int32),
            pltpu.VMEM((latent_dim, FBLK), jnp.float32),
            pltpu.VMEM((latent_dim, FBLK), jnp.float32),
            pltpu.VMEM((latent_dim,), jnp.float32),
            pltpu.VMEM((latent_dim,), jnp.float32),
            pltpu.VMEM((b_per_w, latent_dim), jnp.float32),
            pltpu.SemaphoreType.DMA,
            pltpu.SemaphoreType.DMA,
        ],
        compiler_params=pltpu.CompilerParams(needs_layout_passes=False),
    )
    return run(style_ids, frame_ids, lat_t, style_latents_mu)


# speedup vs baseline: 2.7929x; 1.6240x over previous
import functools

import jax
import jax.numpy as jnp
from jax import lax
from jax.experimental import pallas as pl
from jax.experimental.pallas import tpu as pltpu
from jax.experimental.pallas import tpu_sc as plsc

SIGMA = 1.0
NUM_CORES = 2
NUM_SUBCORES = 16
NUM_WORKERS = NUM_CORES * NUM_SUBCORES
LANES = 16
FBLK = 128
NBUF = 6


def _body(latent_dim, b_per_w, style_hbm, frame_hbm, lat_hbm, mu_hbm,
          out_hbm, sid_s, fid_s, slabs, murs, out_v, sems):
    wid = lax.axis_index("s") * NUM_CORES + lax.axis_index("c")
    base = wid * b_per_w

    pltpu.sync_copy(style_hbm.at[pl.ds(base, b_per_w)],
                    sid_s.at[pl.ds(0, b_per_w)])
    pltpu.sync_copy(frame_hbm.at[pl.ds(base, b_per_w)],
                    fid_s.at[pl.ds(0, b_per_w)])

    def sid(b):
        return sid_s[pl.ds(b, LANES)][0]

    def fid(b):
        return fid_s[pl.ds(b, LANES)][0]

    def slab_src(b):
        off = pl.multiple_of((fid(b) // FBLK) * FBLK, FBLK)
        return lat_hbm.at[sid(b), :, pl.ds(off, FBLK)]

    def mu_src(b):
        return mu_hbm.at[sid(b), :]

    def fire(b, k):
        pltpu.async_copy(slab_src(b), slabs[k], sems[k])
        pltpu.async_copy(mu_src(b), murs[k], sems[k])

    def wait(b, k):
        pltpu.make_async_copy(slab_src(b), slabs[k], sems[k]).wait()
        pltpu.make_async_copy(mu_src(b), murs[k], sems[k]).wait()

    for b in range(NBUF - 1):
        fire(b, b % NBUF)

    d_iota = lax.broadcasted_iota(jnp.int32, (LANES,), 0)

    def step(b, _):
        nxt = b + NBUF - 1

        @pl.when(nxt < b_per_w)
        def _prefetch():
            for k in range(NBUF):
                @pl.when(nxt % NBUF == k)
                def _(k=k):
                    fire(nxt, k)

        foff = fid(b) % FBLK
        col_idx = jnp.full((LANES,), foff, jnp.int32)

        for k in range(NBUF):
            @pl.when(b % NBUF == k)
            def _consume(k=k):
                wait(b, k)
                for d0 in range(0, latent_dim, LANES):
                    lat = plsc.load_gather(slabs[k], [d0 + d_iota, col_idx])
                    mu = murs[k][pl.ds(d0, LANES)]
                    out_v[b, pl.ds(d0, LANES)] = mu + SIGMA * (lat - mu)

        return 0

    lax.fori_loop(0, b_per_w, step, 0)

    pltpu.sync_copy(out_v, out_hbm.at[pl.ds(base, b_per_w)])


def kernel(style_ids, frame_ids, latents, style_latents_mu):
    style_num, frame_num, latent_dim = latents.shape
    batch = style_ids.shape[0]
    b_per_w = batch // NUM_WORKERS
    lat_t = jnp.transpose(latents, (0, 2, 1))

    mesh = plsc.VectorSubcoreMesh(core_axis_name="c", subcore_axis_name="s",
                                  num_cores=NUM_CORES,
                                  num_subcores=NUM_SUBCORES)
    run = pl.kernel(
        functools.partial(_body, latent_dim, b_per_w),
        out_type=jax.ShapeDtypeStruct((batch, latent_dim), jnp.float32),
        mesh=mesh,
        scratch_types=[
            pltpu.VMEM((b_per_w + LANES,), jnp.int32),
            pltpu.VMEM((b_per_w + LANES,), jnp.int32),
            [pltpu.VMEM((latent_dim, FBLK), jnp.float32)
             for _ in range(NBUF)],
            [pltpu.VMEM((latent_dim,), jnp.float32)
             for _ in range(NBUF)],
            pltpu.VMEM((b_per_w, latent_dim), jnp.float32),
            [pltpu.SemaphoreType.DMA for _ in range(NBUF)],
        ],
        compiler_params=pltpu.CompilerParams(needs_layout_passes=False),
    )
    return run(style_ids, frame_ids, lat_t, style_latents_mu)


# speedup vs baseline: 2.9239x; 1.0469x over previous
import functools

import jax
import jax.numpy as jnp
from jax import lax
from jax.experimental import pallas as pl
from jax.experimental.pallas import tpu as pltpu
from jax.experimental.pallas import tpu_sc as plsc

SIGMA = 1.0
NUM_CORES = 2
NUM_SUBCORES = 16
NUM_WORKERS = NUM_CORES * NUM_SUBCORES
LANES = 16
FBLK = 128
NBUF = 7


def _body(latent_dim, b_per_w, style_hbm, frame_hbm, lat_hbm, mu_hbm,
          out_hbm, sid_s, fid_s, slabs, murs, out_v, sems):
    wid = lax.axis_index("s") * NUM_CORES + lax.axis_index("c")
    base = wid * b_per_w

    pltpu.sync_copy(style_hbm.at[pl.ds(base, b_per_w)],
                    sid_s.at[pl.ds(0, b_per_w)])
    pltpu.sync_copy(frame_hbm.at[pl.ds(base, b_per_w)],
                    fid_s.at[pl.ds(0, b_per_w)])

    def sid(b):
        return sid_s[pl.ds(b, LANES)][0]

    def fid(b):
        return fid_s[pl.ds(b, LANES)][0]

    def slab_src(b):
        off = pl.multiple_of((fid(b) // FBLK) * FBLK, FBLK)
        return lat_hbm.at[sid(b), :, pl.ds(off, FBLK)]

    def mu_src(b):
        return mu_hbm.at[sid(b), :]

    def fire(b, k):
        pltpu.async_copy(slab_src(b), slabs[k], sems[k])
        pltpu.async_copy(mu_src(b), murs[k], sems[k])

    def wait(b, k):
        pltpu.make_async_copy(slab_src(b), slabs[k], sems[k]).wait()
        pltpu.make_async_copy(mu_src(b), murs[k], sems[k]).wait()

    for b in range(NBUF - 1):
        fire(b, b % NBUF)

    d_iota = lax.broadcasted_iota(jnp.int32, (LANES,), 0)

    def step(b, _):
        nxt = b + NBUF - 1

        @pl.when(nxt < b_per_w)
        def _prefetch():
            for k in range(NBUF):
                @pl.when(nxt % NBUF == k)
                def _(k=k):
                    fire(nxt, k)

        foff = fid(b) % FBLK
        col_idx = jnp.full((LANES,), foff, jnp.int32)

        for k in range(NBUF):
            @pl.when(b % NBUF == k)
            def _consume(k=k):
                wait(b, k)
                for d0 in range(0, latent_dim, LANES):
                    lat = plsc.load_gather(slabs[k], [d0 + d_iota, col_idx])
                    mu = murs[k][pl.ds(d0, LANES)]
                    out_v[b, pl.ds(d0, LANES)] = mu + SIGMA * (lat - mu)

        return 0

    lax.fori_loop(0, b_per_w, step, 0)

    pltpu.sync_copy(out_v, out_hbm.at[pl.ds(base, b_per_w)])


def kernel(style_ids, frame_ids, latents, style_latents_mu):
    style_num, frame_num, latent_dim = latents.shape
    batch = style_ids.shape[0]
    b_per_w = batch // NUM_WORKERS
    lat_t = jnp.transpose(latents, (0, 2, 1))

    mesh = plsc.VectorSubcoreMesh(core_axis_name="c", subcore_axis_name="s",
                                  num_cores=NUM_CORES,
                                  num_subcores=NUM_SUBCORES)
    run = pl.kernel(
        functools.partial(_body, latent_dim, b_per_w),
        out_type=jax.ShapeDtypeStruct((batch, latent_dim), jnp.float32),
        mesh=mesh,
        scratch_types=[
            pltpu.VMEM((b_per_w + LANES,), jnp.int32),
            pltpu.VMEM((b_per_w + LANES,), jnp.int32),
            [pltpu.VMEM((latent_dim, FBLK), jnp.float32)
             for _ in range(NBUF)],
            [pltpu.VMEM((latent_dim,), jnp.float32)
             for _ in range(NBUF)],
            pltpu.VMEM((b_per_w, latent_dim), jnp.float32),
            [pltpu.SemaphoreType.DMA for _ in range(NBUF)],
        ],
        compiler_params=pltpu.CompilerParams(needs_layout_passes=False),
    )
    return run(style_ids, frame_ids, lat_t, style_latents_mu)
